# Initial kernel scaffold; baseline (speedup 1.0000x reference)
import functools

import jax
import jax.numpy as jnp
from jax import lax
from jax.experimental import pallas as pl
from jax.experimental.pallas import tpu as pltpu
from jax.experimental.pallas import tpu_sc as plsc

_N = 10000
_E = 640000
_B = 128
_RB = 1000
_GRID = _N // _RB
_CH = 128
_NCHUNK_PAD = 5008
_CPT = _NCHUNK_PAD // 16
_ZPT = 10016 // 16
_OPT = _N // 16

_NEG_INF = jnp.float32(-jnp.inf)


def _ka(x_ref, wt_ref, b_ref, t_ref, h_ref, m_ref):
    i = pl.program_id(0)
    h = jnp.dot(x_ref[...], wt_ref[...], preferred_element_type=jnp.float32)
    h = jnp.maximum(h + b_ref[...], 0.0) + 1e-7
    h_ref[...] = h
    cm = jnp.max(h * t_ref[0, 0], axis=0, keepdims=True)
    cmb = jnp.broadcast_to(cm, (8, 128))

    @pl.when(i == 0)
    def _():
        m_ref[...] = cmb

    @pl.when(i != 0)
    def _():
        m_ref[...] = jnp.maximum(m_ref[...], cmb)


def _kb(h_ref, m_ref, t_ref, x_ref, wdt_ref, bd_ref, p_ref, q_ref, xd_ref):
    m = jnp.max(m_ref[...], axis=0, keepdims=True)
    h = h_ref[...]
    p = jnp.exp(h * t_ref[0, 0] - m)
    p_ref[...] = p
    q_ref[...] = p * h
    xd = jnp.dot(x_ref[...], wdt_ref[...], preferred_element_type=jnp.float32)
    xd_ref[...] = xd + bd_ref[...]


def _sc_body(p_hbm, q_hbm, src_hbm, dst_hbm, zeros_hbm, out_hbm,
             sidx, didx, rows, acc_sh, sem):
    c = lax.axis_index("c")
    s = lax.axis_index("s")

    pltpu.sync_copy(zeros_hbm, acc_sh.at[pl.ds(s * _ZPT, _ZPT)])
    pltpu.sync_copy(src_hbm.at[pl.ds(s * _CPT, _CPT)], sidx)
    pltpu.sync_copy(dst_hbm.at[pl.ds(s * _CPT, _CPT)], didx)
    plsc.subcore_barrier()

    def run(tab_hbm):
        def body(j, carry):
            pltpu.async_copy(tab_hbm.at[sidx.at[j]], rows, sem).wait()
            pltpu.sync_copy(rows, acc_sh.at[didx.at[j]], add=True)
            return carry

        lax.fori_loop(0, _CPT, body, 0)

    @pl.when(c == 0)
    def _():
        run(p_hbm)

    @pl.when(c == 1)
    def _():
        run(q_hbm)

    plsc.subcore_barrier()
    pltpu.sync_copy(acc_sh.at[pl.ds(s * _OPT, _OPT)],
                    out_hbm.at[pl.ds(c * _N + s * _OPT, _OPT)])


_sc_aggregate = functools.partial(
    pl.kernel,
    out_type=jax.ShapeDtypeStruct((2 * _N, 128), jnp.float32),
    mesh=plsc.VectorSubcoreMesh(core_axis_name="c", subcore_axis_name="s"),
    scratch_types=[
        pltpu.VMEM((_CPT, _CH), jnp.int32),
        pltpu.VMEM((_CPT, _CH), jnp.int32),
        pltpu.VMEM((_CH, 128), jnp.float32),
        pltpu.VMEM_SHARED((10016, 128), jnp.float32),
        pltpu.SemaphoreType.DMA,
    ],
)(_sc_body)


def _kc(dn_ref, nm_ref, xd_ref, w1_ref, b1_ref, g_ref, be_ref,
        w2_ref, b2_ref, sc_ref, sh_ref, bs_ref, bv_ref, pool_ref):
    i = pl.program_id(0)
    out = nm_ref[...] / (dn_ref[...] + 1e-16) + xd_ref[...]
    hm = jnp.dot(out, w1_ref[...], preferred_element_type=jnp.float32)
    hm = hm + b1_ref[...]
    mu = jnp.mean(hm, axis=-1, keepdims=True)
    var = jnp.mean((hm - mu) ** 2, axis=-1, keepdims=True)
    hm = (hm - mu) * lax.rsqrt(var + 1e-5) * g_ref[...] + be_ref[...]
    hm = jnp.maximum(hm, 0.0)
    xg = jnp.dot(hm, w2_ref[...], preferred_element_type=jnp.float32)
    xg = (xg + b2_ref[...]) * sc_ref[...] + sh_ref[...]

    @pl.when(i == 0)
    def _():
        pool_ref[...] = jnp.full((_B, 128), _NEG_INF, jnp.float32)

    bv = bv_ref[0]
    lo = bs_ref[i, 0]
    hi = bs_ref[i, _RB - 1]

    def body(g, carry):
        masked = jnp.where(bv == g, xg, _NEG_INF)
        cm = jnp.max(masked, axis=0, keepdims=True)
        row = pool_ref[pl.ds(g, 1), :]
        pool_ref[pl.ds(g, 1), :] = jnp.maximum(row, cm)
        return carry

    lax.fori_loop(lo, hi + 1, body, 0)


def _kd(pool_ref, prot_ref, f1w_ref, f1b_ref, b3s_ref, b3h_ref,
        f2w_ref, f2b_ref, f3w_ref, f3b_ref, cxw_ref, cxb_ref,
        f11w_ref, f11b_ref, f12w_ref, f12b_ref, ow_ref, ob_ref, o_ref):
    pooled = pool_ref[...]
    pooled = jnp.where(jnp.isfinite(pooled), pooled, 0.0)
    z = jnp.dot(pooled, f1w_ref[...], preferred_element_type=jnp.float32)
    z = jnp.maximum(z + f1b_ref[...], 0.0)
    z = z * b3s_ref[...] + b3h_ref[...]
    z = jnp.dot(z, f2w_ref[...], preferred_element_type=jnp.float32)
    z = jnp.maximum(z + f2b_ref[...], 0.0)
    drugs = jnp.dot(z, f3w_ref[...], preferred_element_type=jnp.float32)
    drugs = drugs + f3b_ref[...]
    conv = jnp.dot(prot_ref[...], cxw_ref[...],
                   preferred_element_type=jnp.float32)
    conv = jnp.maximum(conv + cxb_ref[...], 0.0)
    xc = jnp.concatenate([drugs, conv], axis=1)
    xc = jnp.dot(xc, f11w_ref[...], preferred_element_type=jnp.float32)
    xc = xc + f11b_ref[...]
    xc = jnp.maximum(
        jnp.dot(xc, f12w_ref[...], preferred_element_type=jnp.float32)
        + f12b_ref[...], 0.0)
    o_ref[...] = jnp.dot(xc, ow_ref[...],
                         preferred_element_type=jnp.float32) + ob_ref[...]


def kernel(proteins, edge_index, edge_attr, x, batch, params):
    p = params
    f32 = jnp.float32

    wsrc_t = p['W_src'].T
    b_src = p['b_src'][None, :]
    wdst_t = p['W_dst'].T
    b_dst = p['b_dst'][None, :]
    t = jnp.reshape(p['t'], (1, 1))
    w1_t = p['mlp_W1'].T
    b1 = p['mlp_b1'][None, :]
    ln_g = p['ln_g'][None, :]
    ln_b = p['ln_b'][None, :]
    w2_t = p['mlp_W2'].T
    b2 = p['mlp_b2'][None, :]
    bn1_s = (p['bn1_g'] / jnp.sqrt(p['bn1_v'] + 1e-5))[None, :]
    bn1_h = (p['bn1_b'] - p['bn1_m'] * bn1_s[0])[None, :]
    bn3_s = (p['bn3_g'] / jnp.sqrt(p['bn3_v'] + 1e-5))[None, :]
    bn3_h = (p['bn3_b'] - p['bn3_m'] * bn3_s[0])[None, :]

    h, m8 = pl.pallas_call(
        _ka,
        grid=(_GRID,),
        in_specs=[
            pl.BlockSpec((_RB, 32), lambda i: (i, 0)),
            pl.BlockSpec((32, 128), lambda i: (0, 0)),
            pl.BlockSpec((1, 128), lambda i: (0, 0)),
            pl.BlockSpec((1, 1), lambda i: (0, 0)),
        ],
        out_specs=[
            pl.BlockSpec((_RB, 128), lambda i: (i, 0)),
            pl.BlockSpec((8, 128), lambda i: (0, 0)),
        ],
        out_shape=[
            jax.ShapeDtypeStruct((_N, 128), f32),
            jax.ShapeDtypeStruct((8, 128), f32),
        ],
    )(x, wsrc_t, b_src, t)

    pt, qt, xd = pl.pallas_call(
        _kb,
        grid=(_GRID,),
        in_specs=[
            pl.BlockSpec((_RB, 128), lambda i: (i, 0)),
            pl.BlockSpec((8, 128), lambda i: (0, 0)),
            pl.BlockSpec((1, 1), lambda i: (0, 0)),
            pl.BlockSpec((_RB, 32), lambda i: (i, 0)),
            pl.BlockSpec((32, 128), lambda i: (0, 0)),
            pl.BlockSpec((1, 128), lambda i: (0, 0)),
        ],
        out_specs=[
            pl.BlockSpec((_RB, 128), lambda i: (i, 0)),
            pl.BlockSpec((_RB, 128), lambda i: (i, 0)),
            pl.BlockSpec((_RB, 128), lambda i: (i, 0)),
        ],
        out_shape=[
            jax.ShapeDtypeStruct((_N, 128), f32),
            jax.ShapeDtypeStruct((_N, 128), f32),
            jax.ShapeDtypeStruct((_N, 128), f32),
        ],
    )(h, m8, t, x, wdst_t, b_dst)

    e_pad = _NCHUNK_PAD * _CH
    src = jnp.concatenate(
        [edge_index[0], jnp.zeros((e_pad - _E,), jnp.int32)]
    ).reshape(_NCHUNK_PAD, _CH)
    dst = jnp.concatenate(
        [edge_index[1], jnp.full((e_pad - _E,), _N, jnp.int32)]
    ).reshape(_NCHUNK_PAD, _CH)
    zeros = jnp.zeros((_ZPT, 128), f32)

    acc = _sc_aggregate(pt, qt, src, dst, zeros)
    den = acc[:_N]
    num = acc[_N:]

    batch2 = batch.reshape(_GRID, _RB)
    batch3 = batch.reshape(_GRID, _RB, 1)
    pooled = pl.pallas_call(
        _kc,
        grid=(_GRID,),
        in_specs=[
            pl.BlockSpec((_RB, 128), lambda i: (i, 0)),
            pl.BlockSpec((_RB, 128), lambda i: (i, 0)),
            pl.BlockSpec((_RB, 128), lambda i: (i, 0)),
            pl.BlockSpec((128, 256), lambda i: (0, 0)),
            pl.BlockSpec((1, 256), lambda i: (0, 0)),
            pl.BlockSpec((1, 256), lambda i: (0, 0)),
            pl.BlockSpec((1, 256), lambda i: (0, 0)),
            pl.BlockSpec((256, 128), lambda i: (0, 0)),
            pl.BlockSpec((1, 128), lambda i: (0, 0)),
            pl.BlockSpec((1, 128), lambda i: (0, 0)),
            pl.BlockSpec((1, 128), lambda i: (0, 0)),
            pl.BlockSpec(memory_space=pltpu.SMEM),
            pl.BlockSpec((1, _RB, 1), lambda i: (i, 0, 0)),
        ],
        out_specs=pl.BlockSpec((_B, 128), lambda i: (0, 0)),
        out_shape=jax.ShapeDtypeStruct((_B, 128), f32),
    )(den, num, xd, w1_t, b1, ln_g, ln_b, w2_t, b2, bn1_s, bn1_h,
      batch2, batch3)

    out = pl.pallas_call(
        _kd,
        in_specs=[
            pl.BlockSpec((_B, 128), lambda: (0, 0)),
            pl.BlockSpec((_B, 32), lambda: (0, 0)),
            pl.BlockSpec((128, 64), lambda: (0, 0)),
            pl.BlockSpec((1, 64), lambda: (0, 0)),
            pl.BlockSpec((1, 64), lambda: (0, 0)),
            pl.BlockSpec((1, 64), lambda: (0, 0)),
            pl.BlockSpec((64, 32), lambda: (0, 0)),
            pl.BlockSpec((1, 32), lambda: (0, 0)),
            pl.BlockSpec((32, 1), lambda: (0, 0)),
            pl.BlockSpec((1, 1), lambda: (0, 0)),
            pl.BlockSpec((32, 63), lambda: (0, 0)),
            pl.BlockSpec((1, 63), lambda: (0, 0)),
            pl.BlockSpec((64, 128), lambda: (0, 0)),
            pl.BlockSpec((1, 128), lambda: (0, 0)),
            pl.BlockSpec((128, 32), lambda: (0, 0)),
            pl.BlockSpec((1, 32), lambda: (0, 0)),
            pl.BlockSpec((32, 1), lambda: (0, 0)),
            pl.BlockSpec((1, 1), lambda: (0, 0)),
        ],
        out_specs=pl.BlockSpec((_B, 1), lambda: (0, 0)),
        out_shape=jax.ShapeDtypeStruct((_B, 1), f32),
    )(pooled, proteins,
      p['fc1_W'].T, p['fc1_b'][None, :], bn3_s, bn3_h,
      p['fc2_W'].T, p['fc2_b'][None, :],
      p['fc3_W'].T, p['fc3_b'][None, :],
      p['convx_W'].T, p['convx_b'][None, :],
      p['fc11_W'].T, p['fc11_b'][None, :],
      p['fc12_W'].T, p['fc12_b'][None, :],
      p['out_W'].T, p['out_b'][None, :])
    return out


# speedup vs baseline: 7.7198x; 7.7198x over previous
import functools

import jax
import jax.numpy as jnp
from jax import lax
from jax.experimental import pallas as pl
from jax.experimental.pallas import tpu as pltpu
from jax.experimental.pallas import tpu_sc as plsc

_N = 10000
_E = 640000
_B = 128
_RB = 1000
_GRID = _N // _RB
_CH = 128
_NCHUNK_PAD = 5120
_CPT = _NCHUNK_PAD // 16
_NACC = 10112
_ZPT = _NACC // 16

_NEG_INF = float("-inf")


def _ka(x_ref, wt_ref, b_ref, t_ref, h_ref, m_ref):
    i = pl.program_id(0)
    h = jnp.dot(x_ref[...], wt_ref[...], preferred_element_type=jnp.float32)
    h = jnp.maximum(h + b_ref[...], 0.0) + 1e-7
    h_ref[...] = h
    cm = jnp.max(h * t_ref[0, 0], axis=0, keepdims=True)
    cmb = jnp.broadcast_to(cm, (8, 128))

    @pl.when(i == 0)
    def _():
        m_ref[...] = cmb

    @pl.when(i != 0)
    def _():
        m_ref[...] = jnp.maximum(m_ref[...], cmb)


def _kb(h_ref, m_ref, t_ref, x_ref, wdt_ref, bd_ref, p_ref, q_ref, xd_ref):
    m = jnp.max(m_ref[...], axis=0, keepdims=True)
    h = h_ref[...]
    p = jnp.exp(h * t_ref[0, 0] - m)
    p_ref[...] = p
    q_ref[...] = p * h
    xd = jnp.dot(x_ref[...], wdt_ref[...], preferred_element_type=jnp.float32)
    xd_ref[...] = xd + bd_ref[...]


def _sc_body(p_hbm, q_hbm, src_hbm, dst_hbm, zeros_hbm, out_hbm,
             rowidx, sidx, didx, rows, acc_sh, sem):
    c = lax.axis_index("c")
    s = lax.axis_index("s")
    base = s * _CPT
    iota = lax.iota(jnp.int32, 16)
    for g in range(3):
        for j in range(8):
            rowidx[g, pl.ds(j * 16, 16)] = jnp.minimum(
                iota + (base + g * 128 + j * 16), _NCHUNK_PAD - 1)
    pltpu.sync_copy(zeros_hbm, acc_sh.at[pl.ds(s * _ZPT, _ZPT)])
    plsc.subcore_barrier()

    def run(tab_hbm):
        for g in range(3):
            n = min(128, _CPT - g * 128)
            pltpu.async_copy(src_hbm.at[rowidx.at[g]], sidx, sem).wait()
            pltpu.async_copy(dst_hbm.at[rowidx.at[g]], didx, sem).wait()

            def body(j, carry):
                pltpu.async_copy(tab_hbm.at[sidx.at[j]], rows, sem).wait()
                pltpu.sync_copy(rows, acc_sh.at[didx.at[j]], add=True)
                return carry

            lax.fori_loop(0, n, body, 0)

    @pl.when(c == 0)
    def _():
        run(p_hbm)

    @pl.when(c == 1)
    def _():
        run(q_hbm)

    plsc.subcore_barrier()
    pltpu.sync_copy(acc_sh.at[pl.ds(s * _ZPT, _ZPT)],
                    out_hbm.at[pl.ds(c * _NACC + s * _ZPT, _ZPT)])


_sc_aggregate = functools.partial(
    pl.kernel,
    out_type=jax.ShapeDtypeStruct((2 * _NACC, 128), jnp.float32),
    mesh=plsc.VectorSubcoreMesh(core_axis_name="c", subcore_axis_name="s"),
    scratch_types=[
        pltpu.VMEM((3, 128), jnp.int32),
        pltpu.VMEM((_CH, _CH), jnp.int32),
        pltpu.VMEM((_CH, _CH), jnp.int32),
        pltpu.VMEM((_CH, 128), jnp.float32),
        pltpu.VMEM_SHARED((_NACC, 128), jnp.float32),
        pltpu.SemaphoreType.DMA,
    ],
)(_sc_body)


def _kc(dn_ref, nm_ref, xd_ref, w1_ref, b1_ref, g_ref, be_ref,
        w2_ref, b2_ref, sc_ref, sh_ref, bs_ref, bv_ref, pool_ref):
    i = pl.program_id(0)
    out = nm_ref[...] / (dn_ref[...] + 1e-16) + xd_ref[...]
    hm = jnp.dot(out, w1_ref[...], preferred_element_type=jnp.float32)
    hm = hm + b1_ref[...]
    mu = jnp.mean(hm, axis=-1, keepdims=True)
    var = jnp.mean((hm - mu) ** 2, axis=-1, keepdims=True)
    hm = (hm - mu) * lax.rsqrt(var + 1e-5) * g_ref[...] + be_ref[...]
    hm = jnp.maximum(hm, 0.0)
    xg = jnp.dot(hm, w2_ref[...], preferred_element_type=jnp.float32)
    xg = (xg + b2_ref[...]) * sc_ref[...] + sh_ref[...]

    @pl.when(i == 0)
    def _():
        pool_ref[...] = jnp.full((_B, 128), _NEG_INF, jnp.float32)

    bv = bv_ref[0]
    lo = bs_ref[i, 0]
    hi = bs_ref[i, _RB - 1]

    def body(g, carry):
        masked = jnp.where(bv == g, xg, _NEG_INF)
        cm = jnp.max(masked, axis=0, keepdims=True)
        row = pool_ref[pl.ds(g, 1), :]
        pool_ref[pl.ds(g, 1), :] = jnp.maximum(row, cm)
        return carry

    lax.fori_loop(lo, hi + 1, body, 0)


def _kd(pool_ref, prot_ref, f1w_ref, f1b_ref, b3s_ref, b3h_ref,
        f2w_ref, f2b_ref, f3w_ref, f3b_ref, cxw_ref, cxb_ref,
        f11w_ref, f11b_ref, f12w_ref, f12b_ref, ow_ref, ob_ref, o_ref):
    pooled = pool_ref[...]
    pooled = jnp.where(jnp.isfinite(pooled), pooled, 0.0)
    z = jnp.dot(pooled, f1w_ref[...], preferred_element_type=jnp.float32)
    z = jnp.maximum(z + f1b_ref[...], 0.0)
    z = z * b3s_ref[...] + b3h_ref[...]
    z = jnp.dot(z, f2w_ref[...], preferred_element_type=jnp.float32)
    z = jnp.maximum(z + f2b_ref[...], 0.0)
    drugs = jnp.dot(z, f3w_ref[...], preferred_element_type=jnp.float32)
    drugs = drugs + f3b_ref[...]
    conv = jnp.dot(prot_ref[...], cxw_ref[...],
                   preferred_element_type=jnp.float32)
    conv = jnp.maximum(conv + cxb_ref[...], 0.0)
    xc = jnp.concatenate([drugs, conv], axis=1)
    xc = jnp.dot(xc, f11w_ref[...], preferred_element_type=jnp.float32)
    xc = xc + f11b_ref[...]
    xc = jnp.maximum(
        jnp.dot(xc, f12w_ref[...], preferred_element_type=jnp.float32)
        + f12b_ref[...], 0.0)
    o_ref[...] = jnp.dot(xc, ow_ref[...],
                         preferred_element_type=jnp.float32) + ob_ref[...]


def kernel(proteins, edge_index, edge_attr, x, batch, params):
    p = params
    f32 = jnp.float32

    wsrc_t = p['W_src'].T
    b_src = p['b_src'][None, :]
    wdst_t = p['W_dst'].T
    b_dst = p['b_dst'][None, :]
    t = jnp.reshape(p['t'], (1, 1))
    w1_t = p['mlp_W1'].T
    b1 = p['mlp_b1'][None, :]
    ln_g = p['ln_g'][None, :]
    ln_b = p['ln_b'][None, :]
    w2_t = p['mlp_W2'].T
    b2 = p['mlp_b2'][None, :]
    bn1_s = (p['bn1_g'] / jnp.sqrt(p['bn1_v'] + 1e-5))[None, :]
    bn1_h = (p['bn1_b'] - p['bn1_m'] * bn1_s[0])[None, :]
    bn3_s = (p['bn3_g'] / jnp.sqrt(p['bn3_v'] + 1e-5))[None, :]
    bn3_h = (p['bn3_b'] - p['bn3_m'] * bn3_s[0])[None, :]

    h, m8 = pl.pallas_call(
        _ka,
        grid=(_GRID,),
        in_specs=[
            pl.BlockSpec((_RB, 32), lambda i: (i, 0)),
            pl.BlockSpec((32, 128), lambda i: (0, 0)),
            pl.BlockSpec((1, 128), lambda i: (0, 0)),
            pl.BlockSpec((1, 1), lambda i: (0, 0)),
        ],
        out_specs=[
            pl.BlockSpec((_RB, 128), lambda i: (i, 0)),
            pl.BlockSpec((8, 128), lambda i: (0, 0)),
        ],
        out_shape=[
            jax.ShapeDtypeStruct((_N, 128), f32),
            jax.ShapeDtypeStruct((8, 128), f32),
        ],
    )(x, wsrc_t, b_src, t)

    pt, qt, xd = pl.pallas_call(
        _kb,
        grid=(_GRID,),
        in_specs=[
            pl.BlockSpec((_RB, 128), lambda i: (i, 0)),
            pl.BlockSpec((8, 128), lambda i: (0, 0)),
            pl.BlockSpec((1, 1), lambda i: (0, 0)),
            pl.BlockSpec((_RB, 32), lambda i: (i, 0)),
            pl.BlockSpec((32, 128), lambda i: (0, 0)),
            pl.BlockSpec((1, 128), lambda i: (0, 0)),
        ],
        out_specs=[
            pl.BlockSpec((_RB, 128), lambda i: (i, 0)),
            pl.BlockSpec((_RB, 128), lambda i: (i, 0)),
            pl.BlockSpec((_RB, 128), lambda i: (i, 0)),
        ],
        out_shape=[
            jax.ShapeDtypeStruct((_N, 128), f32),
            jax.ShapeDtypeStruct((_N, 128), f32),
            jax.ShapeDtypeStruct((_N, 128), f32),
        ],
    )(h, m8, t, x, wdst_t, b_dst)

    e_pad = _NCHUNK_PAD * _CH
    src = jnp.concatenate(
        [edge_index[0], jnp.zeros((e_pad - _E,), jnp.int32)]
    ).reshape(_NCHUNK_PAD, _CH)
    dst = jnp.concatenate(
        [edge_index[1], jnp.full((e_pad - _E,), _N, jnp.int32)]
    ).reshape(_NCHUNK_PAD, _CH)
    zeros = jnp.zeros((_ZPT, 128), f32)

    acc = _sc_aggregate(pt, qt, src, dst, zeros)
    den = acc[:_N]
    num = acc[_NACC:_NACC + _N]

    batch2 = batch.reshape(_GRID, _RB)
    batch3 = batch.reshape(_GRID, _RB, 1)
    pooled = pl.pallas_call(
        _kc,
        grid=(_GRID,),
        in_specs=[
            pl.BlockSpec((_RB, 128), lambda i: (i, 0)),
            pl.BlockSpec((_RB, 128), lambda i: (i, 0)),
            pl.BlockSpec((_RB, 128), lambda i: (i, 0)),
            pl.BlockSpec((128, 256), lambda i: (0, 0)),
            pl.BlockSpec((1, 256), lambda i: (0, 0)),
            pl.BlockSpec((1, 256), lambda i: (0, 0)),
            pl.BlockSpec((1, 256), lambda i: (0, 0)),
            pl.BlockSpec((256, 128), lambda i: (0, 0)),
            pl.BlockSpec((1, 128), lambda i: (0, 0)),
            pl.BlockSpec((1, 128), lambda i: (0, 0)),
            pl.BlockSpec((1, 128), lambda i: (0, 0)),
            pl.BlockSpec(memory_space=pltpu.SMEM),
            pl.BlockSpec((1, _RB, 1), lambda i: (i, 0, 0)),
        ],
        out_specs=pl.BlockSpec((_B, 128), lambda i: (0, 0)),
        out_shape=jax.ShapeDtypeStruct((_B, 128), f32),
    )(den, num, xd, w1_t, b1, ln_g, ln_b, w2_t, b2, bn1_s, bn1_h,
      batch2, batch3)

    out = pl.pallas_call(
        _kd,
        in_specs=[
            pl.BlockSpec((_B, 128), lambda: (0, 0)),
            pl.BlockSpec((_B, 32), lambda: (0, 0)),
            pl.BlockSpec((128, 64), lambda: (0, 0)),
            pl.BlockSpec((1, 64), lambda: (0, 0)),
            pl.BlockSpec((1, 64), lambda: (0, 0)),
            pl.BlockSpec((1, 64), lambda: (0, 0)),
            pl.BlockSpec((64, 32), lambda: (0, 0)),
            pl.BlockSpec((1, 32), lambda: (0, 0)),
            pl.BlockSpec((32, 1), lambda: (0, 0)),
            pl.BlockSpec((1, 1), lambda: (0, 0)),
            pl.BlockSpec((32, 63), lambda: (0, 0)),
            pl.BlockSpec((1, 63), lambda: (0, 0)),
            pl.BlockSpec((64, 128), lambda: (0, 0)),
            pl.BlockSpec((1, 128), lambda: (0, 0)),
            pl.BlockSpec((128, 32), lambda: (0, 0)),
            pl.BlockSpec((1, 32), lambda: (0, 0)),
            pl.BlockSpec((32, 1), lambda: (0, 0)),
            pl.BlockSpec((1, 1), lambda: (0, 0)),
        ],
        out_specs=pl.BlockSpec((_B, 1), lambda: (0, 0)),
        out_shape=jax.ShapeDtypeStruct((_B, 1), f32),
    )(pooled, proteins,
      p['fc1_W'].T, p['fc1_b'][None, :], bn3_s, bn3_h,
      p['fc2_W'].T, p['fc2_b'][None, :],
      p['fc3_W'].T, p['fc3_b'][None, :],
      p['convx_W'].T, p['convx_b'][None, :],
      p['fc11_W'].T, p['fc11_b'][None, :],
      p['fc12_W'].T, p['fc12_b'][None, :],
      p['out_W'].T, p['out_b'][None, :])
    return out
